# baseline (speedup 1.0000x reference)
import functools

import jax
import jax.numpy as jnp
from jax import lax
from jax.experimental import pallas as pl
from jax.experimental.pallas import tpu as pltpu


def _rvq_kernel(x_ref, wp_ref, bp_ref, cb_ref, cbh_ref, cbl_ref, cbsq_ref,
                w1_ref, b1_ref, w2_ref, b2_ref, w3_ref, b3_ref,
                ang_ref, idx0_ref, idx1_ref, cl_ref, counts_ref,
                r1_scr, q0_scr, qt_scr,
                *, num_codes, num_blocks):
    i = pl.program_id(0)
    first = i == 0
    last = i == num_blocks + 1
    act0 = i < num_blocks
    act1 = jnp.logical_and(i >= 1, i < num_blocks + 1)
    act2 = i >= 2

    @pl.when(first)
    def _init():
        cl_ref[...] = jnp.zeros_like(cl_ref)
        counts_ref[...] = jnp.zeros_like(counts_ref)

    tt = x_ref.shape[2]
    iota = lax.broadcasted_iota(jnp.int32, (tt, num_codes), 1)
    ones_row = jnp.ones((1, tt), jnp.bfloat16)

    r1_prev = r1_scr[...]
    q0_prev = q0_scr[...]
    qt_prev = qt_scr[...]

    xb = x_ref[0]
    r0 = lax.dot_general(xb, wp_ref[...], (((0,), (0,)), ((), ())),
                         preferred_element_type=jnp.float32) + bp_ref[...]

    cross1 = lax.dot_general(r1_prev, cb_ref[1], (((1,), (1,)), ((), ())),
                             preferred_element_type=jnp.float32)

    hid = jnp.maximum(lax.dot_general(qt_prev, w1_ref[...],
                                      (((1,), (0,)), ((), ())),
                                      preferred_element_type=jnp.float32)
                      + b1_ref[...], 0.0)
    hid = jnp.maximum(lax.dot_general(hid, w2_ref[...],
                                      (((1,), (0,)), ((), ())),
                                      preferred_element_type=jnp.float32)
                      + b2_ref[...], 0.0)
    dec = lax.dot_general(hid, w3_ref[...], (((1,), (0,)), ((), ())),
                          preferred_element_type=jnp.float32) + b3_ref[...]

    @pl.when(act2)
    def _write_ang():
        ang_ref[0] = dec.T

    cross0 = lax.dot_general(r0, cb_ref[0], (((1,), (1,)), ((), ())),
                             preferred_element_type=jnp.float32)

    rsq1 = jnp.sum(r1_prev * r1_prev, axis=1, keepdims=True)
    d1 = rsq1 - 2.0 * cross1 + cbsq_ref[1][None, :]
    idx1 = jnp.argmin(d1, axis=1).astype(jnp.int32)

    rsq0 = jnp.sum(r0 * r0, axis=1, keepdims=True)
    d0 = rsq0 - 2.0 * cross0 + cbsq_ref[0][None, :]
    idx0 = jnp.argmin(d0, axis=1).astype(jnp.int32)

    onehot1 = (iota == idx1[:, None]).astype(jnp.float32).astype(jnp.bfloat16)
    q1 = (lax.dot_general(onehot1, cbh_ref[1], (((1,), (0,)), ((), ())),
                          preferred_element_type=jnp.float32)
          + lax.dot_general(onehot1, cbl_ref[1], (((1,), (0,)), ((), ())),
                            preferred_element_type=jnp.float32))
    diff1 = r1_prev - q1

    onehot0 = (iota == idx0[:, None]).astype(jnp.float32).astype(jnp.bfloat16)
    q0 = (lax.dot_general(onehot0, cbh_ref[0], (((1,), (0,)), ((), ())),
                          preferred_element_type=jnp.float32)
          + lax.dot_general(onehot0, cbl_ref[0], (((1,), (0,)), ((), ())),
                            preferred_element_type=jnp.float32))
    diff0 = r0 - q0

    @pl.when(act1)
    def _commit1():
        idx1_ref[0, 0, :] = idx1
        qt_scr[...] = q0_prev + q1
        cl_ref[...] += jnp.sum(diff1 * diff1).reshape(1, 1)
        counts_ref[...] += lax.dot_general(
            ones_row, onehot1, (((1,), (0,)), ((), ())),
            preferred_element_type=jnp.float32)

    @pl.when(act0)
    def _commit0():
        idx0_ref[0, 0, :] = idx0
        r1_scr[...] = diff0
        q0_scr[...] = q0
        cl_ref[...] += jnp.sum(diff0 * diff0).reshape(1, 1)
        counts_ref[...] += lax.dot_general(
            ones_row, onehot0, (((1,), (0,)), ((), ())),
            preferred_element_type=jnp.float32)


def _perp_kernel(counts_ref, perp_ref):
    c = counts_ref[...]
    avg = c / jnp.sum(c)
    perp = jnp.exp(-jnp.sum(avg * jnp.log(avg + 1e-10)))
    perp_ref[...] = perp.reshape(1, 1)


def kernel(x, W_proj, b_proj, codebooks, W1, b1, W2, b2, W3, b3):
    B, C, T = x.shape
    L, K, D = codebooks.shape
    H = W1.shape[1]
    J = W3.shape[1]
    TT = min(512, T)
    bpb = T // TT
    nb = B * bpb
    N = B * T

    cb_bits = lax.bitcast_convert_type(codebooks, jnp.uint32)
    cb_hi_f32 = lax.bitcast_convert_type(
        cb_bits & jnp.uint32(0xFFFF0000), jnp.float32)
    cb_hi = cb_hi_f32.astype(jnp.bfloat16)
    cb_lo = (codebooks - cb_hi_f32).astype(jnp.bfloat16)
    cbsq = jnp.stack([jnp.sum(codebooks[l] ** 2, axis=1) for l in range(L)])

    kern = functools.partial(_rvq_kernel, num_codes=K, num_blocks=nb)

    def blk(lag):
        def imap(i):
            j = jnp.clip(i - lag, 0, nb - 1)
            return (j // bpb, 0, j % bpb)
        return imap

    full = lambda shape: pl.BlockSpec(shape, lambda i: (0,) * len(shape))
    out_shapes = (
        jax.ShapeDtypeStruct((B, J, T), jnp.float32),
        jax.ShapeDtypeStruct((B, 1, T), jnp.int32),
        jax.ShapeDtypeStruct((B, 1, T), jnp.int32),
        jax.ShapeDtypeStruct((1, 1), jnp.float32),
        jax.ShapeDtypeStruct((1, K), jnp.float32),
    )
    out_specs = (
        pl.BlockSpec((1, J, TT), blk(2)),
        pl.BlockSpec((1, 1, TT), blk(0)),
        pl.BlockSpec((1, 1, TT), blk(1)),
        pl.BlockSpec((1, 1), lambda i: (0, 0)),
        pl.BlockSpec((1, K), lambda i: (0, 0)),
    )
    in_specs = (
        pl.BlockSpec((1, C, TT), blk(0)),
        full((C, D)),
        full((1, D)),
        full((L, K, D)),
        full((L, K, D)),
        full((L, K, D)),
        full((L, K)),
        full((D, H)),
        full((1, H)),
        full((H, H)),
        full((1, H)),
        full((H, J)),
        full((1, J)),
    )

    angles, idx0, idx1, cl_sum, counts = pl.pallas_call(
        kern,
        grid=(nb + 2,),
        in_specs=in_specs,
        out_specs=out_specs,
        out_shape=out_shapes,
        scratch_shapes=[
            pltpu.VMEM((TT, D), jnp.float32),
            pltpu.VMEM((TT, D), jnp.float32),
            pltpu.VMEM((TT, D), jnp.float32),
        ],
        compiler_params=pltpu.CompilerParams(
            dimension_semantics=("arbitrary",)),
    )(x, W_proj, b_proj.reshape(1, D), codebooks, cb_hi, cb_lo, cbsq,
      W1, b1.reshape(1, H), W2, b2.reshape(1, H), W3, b3.reshape(1, J))

    perp = pl.pallas_call(
        _perp_kernel,
        out_shape=jax.ShapeDtypeStruct((1, 1), jnp.float32),
    )(counts)

    indices = jnp.concatenate(
        [idx0.reshape(1, N), idx1.reshape(1, N)], axis=0)
    cl = (cl_sum[0, 0] / (N * D)).astype(jnp.float32)
    codebook_loss = cl
    commit_loss = cl
    vq_loss = cl + 0.25 * cl
    perplexity = perp[0, 0]
    return (angles, indices, vq_loss, codebook_loss, commit_loss, perplexity)


# speedup vs baseline: 1.3859x; 1.3859x over previous
import functools

import jax
import jax.numpy as jnp
from jax import lax
from jax.experimental import pallas as pl
from jax.experimental.pallas import tpu as pltpu


def _rvq_kernel(x_ref, wp_ref, bp_ref, cb_ref, cbh_ref, cbl_ref, cbsq_ref,
                w1_ref, b1_ref, w2_ref, b2_ref, w3_ref, b3_ref,
                ang_ref, idx0_ref, idx1_ref, cl_ref, counts_ref,
                r1_scr, q0_scr, qt_scr,
                *, num_codes, num_blocks):
    i = pl.program_id(0)
    first = i == 0
    act0 = i < num_blocks
    act1 = jnp.logical_and(i >= 1, i < num_blocks + 1)
    act2 = i >= 2

    @pl.when(first)
    def _init():
        cl_ref[...] = jnp.zeros_like(cl_ref)
        counts_ref[...] = jnp.zeros_like(counts_ref)

    tt = x_ref.shape[2]
    iota_k = lax.broadcasted_iota(jnp.int32, (num_codes, tt), 0)
    ones_col = jnp.ones((tt, 1), jnp.bfloat16)

    r1_prev = r1_scr[...]
    q0_prev = q0_scr[...]
    qt_prev = qt_scr[...]

    xb = x_ref[0]
    r0 = lax.dot_general(wp_ref[...], xb, (((0,), (0,)), ((), ())),
                         preferred_element_type=jnp.float32) + bp_ref[...]

    cross1 = lax.dot_general(cb_ref[1], r1_prev, (((1,), (0,)), ((), ())),
                             preferred_element_type=jnp.float32)

    hid = jnp.maximum(lax.dot_general(w1_ref[...], qt_prev,
                                      (((0,), (0,)), ((), ())),
                                      preferred_element_type=jnp.float32)
                      + b1_ref[...], 0.0)
    hid = jnp.maximum(lax.dot_general(w2_ref[...], hid,
                                      (((0,), (0,)), ((), ())),
                                      preferred_element_type=jnp.float32)
                      + b2_ref[...], 0.0)
    dec = lax.dot_general(w3_ref[...], hid, (((0,), (0,)), ((), ())),
                          preferred_element_type=jnp.float32) + b3_ref[...]

    @pl.when(act2)
    def _write_ang():
        ang_ref[0] = dec

    cross0 = lax.dot_general(cb_ref[0], r0, (((1,), (0,)), ((), ())),
                             preferred_element_type=jnp.float32)

    rsq1 = jnp.sum(r1_prev * r1_prev, axis=0, keepdims=True)
    d1 = rsq1 - 2.0 * cross1 + cbsq_ref[1]
    idx1 = jnp.argmin(d1, axis=0).astype(jnp.int32)

    rsq0 = jnp.sum(r0 * r0, axis=0, keepdims=True)
    d0 = rsq0 - 2.0 * cross0 + cbsq_ref[0]
    idx0 = jnp.argmin(d0, axis=0).astype(jnp.int32)

    onehot1 = (iota_k == idx1[None, :]).astype(jnp.float32) \
        .astype(jnp.bfloat16)
    q1 = (lax.dot_general(cbh_ref[1], onehot1, (((0,), (0,)), ((), ())),
                          preferred_element_type=jnp.float32)
          + lax.dot_general(cbl_ref[1], onehot1, (((0,), (0,)), ((), ())),
                            preferred_element_type=jnp.float32))
    diff1 = r1_prev - q1

    onehot0 = (iota_k == idx0[None, :]).astype(jnp.float32) \
        .astype(jnp.bfloat16)
    q0 = (lax.dot_general(cbh_ref[0], onehot0, (((0,), (0,)), ((), ())),
                          preferred_element_type=jnp.float32)
          + lax.dot_general(cbl_ref[0], onehot0, (((0,), (0,)), ((), ())),
                            preferred_element_type=jnp.float32))
    diff0 = r0 - q0

    @pl.when(act1)
    def _commit1():
        idx1_ref[0, 0, :] = idx1
        qt_scr[...] = q0_prev + q1
        cl_ref[...] += jnp.sum(diff1 * diff1).reshape(1, 1)
        counts_ref[...] += lax.dot_general(
            onehot1, ones_col, (((1,), (0,)), ((), ())),
            preferred_element_type=jnp.float32)

    @pl.when(act0)
    def _commit0():
        idx0_ref[0, 0, :] = idx0
        r1_scr[...] = diff0
        q0_scr[...] = q0
        cl_ref[...] += jnp.sum(diff0 * diff0).reshape(1, 1)
        counts_ref[...] += lax.dot_general(
            onehot0, ones_col, (((1,), (0,)), ((), ())),
            preferred_element_type=jnp.float32)


def _perp_kernel(counts_ref, perp_ref):
    c = counts_ref[...]
    avg = c / jnp.sum(c)
    perp = jnp.exp(-jnp.sum(avg * jnp.log(avg + 1e-10)))
    perp_ref[...] = perp.reshape(1, 1)


def _impl(x, W_proj, b_proj, codebooks, cb_hi, cb_lo, cbsq,
          W1, b1, W2, b2, W3, b3):
    B, C, T = x.shape
    L, K, D = codebooks.shape
    H = W1.shape[1]
    J = W3.shape[1]
    TT = min(1024, T)
    bpb = T // TT
    nb = B * bpb

    kern = functools.partial(_rvq_kernel, num_codes=K, num_blocks=nb)

    def blk(lag):
        def imap(i):
            j = jnp.clip(i - lag, 0, nb - 1)
            return (j // bpb, 0, j % bpb)
        return imap

    full = lambda shape: pl.BlockSpec(shape, lambda i: (0,) * len(shape))
    out_shapes = (
        jax.ShapeDtypeStruct((B, J, T), jnp.float32),
        jax.ShapeDtypeStruct((B, 1, T), jnp.int32),
        jax.ShapeDtypeStruct((B, 1, T), jnp.int32),
        jax.ShapeDtypeStruct((1, 1), jnp.float32),
        jax.ShapeDtypeStruct((K, 1), jnp.float32),
    )
    out_specs = (
        pl.BlockSpec((1, J, TT), blk(2)),
        pl.BlockSpec((1, 1, TT), blk(0)),
        pl.BlockSpec((1, 1, TT), blk(1)),
        pl.BlockSpec((1, 1), lambda i: (0, 0)),
        pl.BlockSpec((K, 1), lambda i: (0, 0)),
    )
    in_specs = (
        pl.BlockSpec((1, C, TT), blk(0)),
        full((C, D)),
        full((D, 1)),
        full((L, K, D)),
        full((L, K, D)),
        full((L, K, D)),
        full((L, K, 1)),
        full((D, H)),
        full((H, 1)),
        full((H, H)),
        full((H, 1)),
        full((H, J)),
        full((J, 1)),
    )

    angles, idx0, idx1, cl_sum, counts = pl.pallas_call(
        kern,
        grid=(nb + 2,),
        in_specs=in_specs,
        out_specs=out_specs,
        out_shape=out_shapes,
        scratch_shapes=[
            pltpu.VMEM((D, TT), jnp.float32),
            pltpu.VMEM((D, TT), jnp.float32),
            pltpu.VMEM((D, TT), jnp.float32),
        ],
        compiler_params=pltpu.CompilerParams(
            dimension_semantics=("arbitrary",)),
    )(x, W_proj, b_proj.reshape(D, 1), codebooks, cb_hi, cb_lo, cbsq,
      W1, b1.reshape(H, 1), W2, b2.reshape(H, 1), W3, b3.reshape(J, 1))

    perp = pl.pallas_call(
        _perp_kernel,
        out_shape=jax.ShapeDtypeStruct((1, 1), jnp.float32),
    )(counts)
    return angles, idx0, idx1, cl_sum, perp


def kernel(x, W_proj, b_proj, codebooks, W1, b1, W2, b2, W3, b3):
    B, C, T = x.shape
    L, K, D = codebooks.shape
    N = B * T

    cb_bits = lax.bitcast_convert_type(codebooks, jnp.uint32)
    cb_hi_f32 = lax.bitcast_convert_type(
        cb_bits & jnp.uint32(0xFFFF0000), jnp.float32)
    cb_hi = cb_hi_f32.astype(jnp.bfloat16)
    cb_lo = (codebooks - cb_hi_f32).astype(jnp.bfloat16)
    cbsq = jnp.stack(
        [jnp.sum(codebooks[l] ** 2, axis=1) for l in range(L)])[..., None]

    angles, idx0, idx1, cl_sum, perp = _impl(
        x, W_proj, b_proj, codebooks, cb_hi, cb_lo, cbsq,
        W1, b1, W2, b2, W3, b3)

    indices = jnp.concatenate(
        [idx0.reshape(1, N), idx1.reshape(1, N)], axis=0)
    cl = (cl_sum[0, 0] / (N * D)).astype(jnp.float32)
    codebook_loss = cl
    commit_loss = cl
    vq_loss = cl + 0.25 * cl
    perplexity = perp[0, 0]
    return (angles, indices, vq_loss, codebook_loss, commit_loss, perplexity)
